# Initial kernel scaffold; baseline (speedup 1.0000x reference)
import jax
import jax.numpy as jnp
from jax.experimental import pallas as pl


def kernel(kg_emb, ccle, node_id, edge_index, edge_type, ccle_w1, ccle_b1, ccle_w2, ccle_b2, w_rel1, q1, k1, bias1, w_rel2, q2, k2, bias2, skip_w1, skip_b1, skip_w2, skip_b2):
    raise NotImplementedError("write your pallas kernel here")


# speedup vs baseline: 12.0850x; 12.0850x over previous
import functools

import jax
import jax.numpy as jnp
from jax import lax
from jax.experimental import pallas as pl
from jax.experimental.pallas import tpu as pltpu
from jax.experimental.pallas import tpu_sc as plsc

N = 10000
E = 320000
R = 8
H = 4
KG_DIM = 128
IN_CH = 256
HID = 128
OUT = 128

NC = 2
NS = 16
NW = NC * NS
CH = 128
NCHUNK = E // CH
NFULL = NCHUNK // NW
XTRA = NCHUNK - NFULL * NW
NP = 10240
RPS = NP // NS

_B = 1000
_NB = N // _B

f32 = jnp.float32
i32 = jnp.int32


def _leaky(x, s):
    return jnp.where(x >= 0, x, s * x)


def _prep_body(kg_ref, ccle_ref, w1_ref, b1_ref, w2_ref, b2_ref,
               sw1_ref, sb1_ref, sw2_ref, sb2_ref, xin_ref, skip_ref):
    h = jnp.dot(ccle_ref[...], w1_ref[...], preferred_element_type=f32) + b1_ref[...]
    h = _leaky(h, 0.01)
    co = jnp.dot(h, w2_ref[...], preferred_element_type=f32) + b2_ref[...]
    kg = kg_ref[...]
    xin_ref[:, :KG_DIM] = kg
    xin_ref[:, KG_DIM:] = co
    xi = jnp.concatenate([kg, co], axis=1)
    s = jnp.dot(xi, sw1_ref[...], preferred_element_type=f32) + sb1_ref[...]
    s = _leaky(s, 0.01)
    skip_ref[...] = jnp.dot(s, sw2_ref[...], preferred_element_type=f32) + sb2_ref[...]


def _prep(kg, ccle, w1, b1, w2, b2, sw1, sb1, sw2, sb2):
    full = lambda shape: pl.BlockSpec(shape, lambda j: tuple(0 for _ in shape))
    return pl.pallas_call(
        _prep_body,
        grid=(_NB,),
        in_specs=[
            pl.BlockSpec((_B, KG_DIM), lambda j: (j, 0)),
            pl.BlockSpec((_B, 4), lambda j: (j, 0)),
            full((4, 32)), full((32,)), full((32, 128)), full((128,)),
            full((IN_CH, HID)), full((HID,)), full((HID, OUT)), full((OUT,)),
        ],
        out_specs=[
            pl.BlockSpec((_B, IN_CH), lambda j: (j, 0)),
            pl.BlockSpec((_B, OUT), lambda j: (j, 0)),
        ],
        out_shape=[
            jax.ShapeDtypeStruct((N, IN_CH), f32),
            jax.ShapeDtypeStruct((N, OUT), f32),
        ],
    )(kg, ccle, w1, b1, w2, b2, sw1, sb1, sw2, sb2)


def _emit_tables(xb, q_ref, k_ref, xr_ref, aq_ref, ak_ref, pm_ref):
    xr_ref[0] = xb
    aq = jnp.dot(xb, q_ref[...], preferred_element_type=f32)
    ak = jnp.dot(xb, k_ref[...], preferred_element_type=f32)
    pad = jnp.zeros((xb.shape[0], 16 - H), f32)
    aq_ref[0] = jnp.concatenate([aq, pad], axis=1)
    ak_ref[0] = jnp.concatenate([ak, pad], axis=1)
    pm_ref[...] = jnp.concatenate(
        [jnp.max(aq, axis=0), jnp.max(ak, axis=0)]).reshape(1, 1, 2 * H)


def _tables_body(x_ref, wr_ref, q_ref, k_ref, xr_ref, aq_ref, ak_ref, pm_ref):
    xb = jnp.dot(x_ref[...], wr_ref[0], preferred_element_type=f32)
    _emit_tables(xb, q_ref, k_ref, xr_ref, aq_ref, ak_ref, pm_ref)


def _tables_mid_body(p_ref, b_ref, wr_ref, q_ref, k_ref,
                     xr_ref, aq_ref, ak_ref, pm_ref):
    x = _leaky(p_ref[0] + p_ref[1] + b_ref[...], 0.01)
    xb = jnp.dot(x, wr_ref[0], preferred_element_type=f32)
    _emit_tables(xb, q_ref, k_ref, xr_ref, aq_ref, ak_ref, pm_ref)


def _table_out_specs():
    return dict(
        out_specs=[
            pl.BlockSpec((1, _B, HID), lambda r, j: (r, j, 0)),
            pl.BlockSpec((1, _B, 16), lambda r, j: (r, j, 0)),
            pl.BlockSpec((1, _B, 16), lambda r, j: (r, j, 0)),
            pl.BlockSpec((1, 1, 2 * H), lambda r, j: (r * _NB + j, 0, 0)),
        ],
        out_shape=[
            jax.ShapeDtypeStruct((R, N, HID), f32),
            jax.ShapeDtypeStruct((R, N, 16), f32),
            jax.ShapeDtypeStruct((R, N, 16), f32),
            jax.ShapeDtypeStruct((R * _NB, 1, 2 * H), f32),
        ],
    )


def _tables(x, wr, q, k):
    in_ch = x.shape[1]
    return pl.pallas_call(
        _tables_body,
        grid=(R, _NB),
        in_specs=[
            pl.BlockSpec((_B, in_ch), lambda r, j: (j, 0)),
            pl.BlockSpec((1, in_ch, HID), lambda r, j: (r, 0, 0)),
            pl.BlockSpec((HID, H), lambda r, j: (0, 0)),
            pl.BlockSpec((HID, H), lambda r, j: (0, 0)),
        ],
        **_table_out_specs(),
    )(x, wr, q, k)


def _tables_mid(p, b, wr, q, k):
    return pl.pallas_call(
        _tables_mid_body,
        grid=(R, _NB),
        in_specs=[
            pl.BlockSpec((NC, _B, HID), lambda r, j: (0, j, 0)),
            pl.BlockSpec((HID,), lambda r, j: (0,)),
            pl.BlockSpec((1, HID, HID), lambda r, j: (r, 0, 0)),
            pl.BlockSpec((HID, H), lambda r, j: (0, 0)),
            pl.BlockSpec((HID, H), lambda r, j: (0, 0)),
        ],
        **_table_out_specs(),
    )(p, b, wr, q, k)


def _mvec(pm):
    mx = jnp.max(pm.reshape(-1, 2 * H), axis=0)
    m4 = _leaky(mx[:H] + mx[H:], 0.2)
    return jnp.concatenate([m4, jnp.zeros((16 - H,), f32)])


def _add2_body(p_ref, o_ref):
    o_ref[...] = p_ref[0] + p_ref[1]


def _add2(p):
    bp = NP // 10
    return pl.pallas_call(
        _add2_body,
        grid=(10,),
        in_specs=[pl.BlockSpec((NC, bp, 16), lambda j: (0, j, 0))],
        out_specs=pl.BlockSpec((bp, 16), lambda j: (j, 0)),
        out_shape=jax.ShapeDtypeStruct((NP, 16), f32),
    )(p)


def _final_body(p_ref, b_ref, skip_ref, o_ref):
    o_ref[...] = _leaky(p_ref[0] + p_ref[1] + b_ref[...] + skip_ref[...], 0.01)


def _final(p, b, skip):
    return pl.pallas_call(
        _final_body,
        grid=(_NB,),
        in_specs=[
            pl.BlockSpec((NC, _B, OUT), lambda j: (0, j, 0)),
            pl.BlockSpec((OUT,), lambda j: (0,)),
            pl.BlockSpec((_B, OUT), lambda j: (j, 0)),
        ],
        out_specs=pl.BlockSpec((_B, OUT), lambda j: (j, 0)),
        out_shape=jax.ShapeDtypeStruct((N, OUT), f32),
    )(p, b, skip)


_MESH = plsc.VectorSubcoreMesh(core_axis_name="c", subcore_axis_name="s")
_SC_PARAMS = pltpu.CompilerParams(use_tc_tiling_on_sc=False)


def _worker_id():
    return lax.axis_index("s") * NC + lax.axis_index("c")


def _nchunks(wid):
    return jnp.where(wid < XTRA, NFULL + 1, NFULL).astype(i32)


def _load_edges(i, wid, et_h, src_h, dst_h, et_v, src_v, dst2d):
    ch = wid + i * NW
    base = ch * CH
    pltpu.sync_copy(et_h.at[pl.ds(base, CH)], et_v)
    pltpu.sync_copy(src_h.at[pl.ds(base, CH)], src_v)
    pltpu.sync_copy(dst_h.at[pl.ds(base, CH)], dst2d.at[0])
    return base


@functools.partial(
    pl.kernel,
    out_type=(
        jax.ShapeDtypeStruct((E, 16), f32),
        jax.ShapeDtypeStruct((NC, NP, 16), f32),
    ),
    mesh=_MESH,
    scratch_types=(
        pltpu.VMEM((CH,), i32),
        pltpu.VMEM((CH,), i32),
        pltpu.VMEM((1, CH), i32),
        pltpu.VMEM((CH,), i32),
        pltpu.VMEM((CH,), i32),
        pltpu.VMEM((CH, 16), f32),
        pltpu.VMEM((CH, 16), f32),
        pltpu.VMEM((CH, 16), f32),
        pltpu.VMEM((16,), f32),
        pltpu.VMEM_SHARED((NP, 16), f32),
    ),
    compiler_params=_SC_PARAMS,
)
def _ab_kernel(et_h, src_h, dst_h, aq_h, ak_h, m_h, z16_h, ex_h, dp_h,
               et_v, src_v, dst2d, idxi, idxj, ai, aj, ex_v, m_v, acc_s):
    cid = lax.axis_index("c")
    sid = lax.axis_index("s")
    wid = _worker_id()
    pltpu.sync_copy(m_h, m_v)
    pltpu.sync_copy(z16_h.at[pl.ds(sid * RPS, RPS), :],
                    acc_s.at[pl.ds(sid * RPS, RPS), :])
    plsc.subcore_barrier()
    mv = m_v[...]

    def body(i, carry):
        base = _load_edges(i, wid, et_h, src_h, dst_h, et_v, src_v, dst2d)
        for v in range(CH // 16):
            sl = pl.ds(v * 16, 16)
            e = et_v[sl] * N
            idxi[sl] = e + dst2d[0, sl]
            idxj[sl] = e + src_v[sl]
        pltpu.sync_copy(aq_h.at[idxi], ai)
        pltpu.sync_copy(ak_h.at[idxj], aj)

        def erow(r, c):
            a = ai[r, :] + aj[r, :]
            a = jnp.where(a >= 0, a, 0.2 * a)
            ex_v[r, :] = jnp.exp(a - mv)
            return c
        lax.fori_loop(0, CH, erow, 0, unroll=4)
        pltpu.sync_copy(ex_v, ex_h.at[pl.ds(base, CH), :])
        pltpu.sync_copy(ex_v, acc_s.at[dst2d.at[0]], add=True)
        return carry

    lax.fori_loop(0, _nchunks(wid), body, 0)
    plsc.subcore_barrier()
    pltpu.sync_copy(acc_s.at[pl.ds(sid * RPS, RPS), :],
                    dp_h.at[cid, pl.ds(sid * RPS, RPS), :])


@functools.partial(
    pl.kernel,
    out_type=jax.ShapeDtypeStruct((NC, NP, HID), f32),
    mesh=_MESH,
    scratch_types=(
        pltpu.VMEM((CH,), i32),
        pltpu.VMEM((CH,), i32),
        pltpu.VMEM((1, CH), i32),
        pltpu.VMEM((CH,), i32),
        pltpu.VMEM((CH, 16), f32),
        pltpu.VMEM((CH, 16), f32),
        pltpu.VMEM((CH, HID), f32),
        pltpu.VMEM_SHARED((NP, HID), f32),
    ),
    compiler_params=_SC_PARAMS,
)
def _msg_kernel(et_h, src_h, dst_h, exb_h, den_h, xr_h, z128_h, op_h,
                et_v, src_v, dst2d, idxj, ex_v, den_v, xr_v, acc_s):
    cid = lax.axis_index("c")
    sid = lax.axis_index("s")
    wid = _worker_id()
    pltpu.sync_copy(z128_h.at[pl.ds(sid * RPS, RPS), :],
                    acc_s.at[pl.ds(sid * RPS, RPS), :])
    plsc.subcore_barrier()

    def body(i, carry):
        base = _load_edges(i, wid, et_h, src_h, dst_h, et_v, src_v, dst2d)
        for v in range(CH // 16):
            sl = pl.ds(v * 16, 16)
            idxj[sl] = et_v[sl] * N + src_v[sl]
        pltpu.sync_copy(exb_h.at[pl.ds(base, CH), :], ex_v)
        pltpu.sync_copy(den_h.at[dst2d.at[0]], den_v)
        pltpu.sync_copy(xr_h.at[idxj], xr_v)

        def erow(r, c):
            at = ex_v[r, :] / (den_v[r, :] + 1e-16)
            for v in range(HID // 16):
                sl = pl.ds(v * 16, 16)
                xr_v[r, sl] = xr_v[r, sl] * at[v * 16 // 32]
            return c
        lax.fori_loop(0, CH, erow, 0, unroll=2)
        pltpu.sync_copy(xr_v, acc_s.at[dst2d.at[0]], add=True)
        return carry

    lax.fori_loop(0, _nchunks(wid), body, 0)
    plsc.subcore_barrier()
    pltpu.sync_copy(acc_s.at[pl.ds(sid * RPS, RPS), :],
                    op_h.at[cid, pl.ds(sid * RPS, RPS), :])


def kernel(kg_emb, ccle, node_id, edge_index, edge_type,
           ccle_w1, ccle_b1, ccle_w2, ccle_b2,
           w_rel1, q1, k1, bias1, w_rel2, q2, k2, bias2,
           skip_w1, skip_b1, skip_w2, skip_b2):
    src = edge_index[0]
    dst = edge_index[1]
    et = edge_type
    z16 = jnp.zeros((NP, 16), f32)
    z128 = jnp.zeros((NP, HID), f32)

    xin, skip = _prep(kg_emb, ccle, ccle_w1, ccle_b1, ccle_w2, ccle_b2,
                      skip_w1, skip_b1, skip_w2, skip_b2)

    xr1, aq1, ak1, pm1 = _tables(xin, w_rel1, q1, k1)
    ex1, dp1 = _ab_kernel(et, src, dst, aq1.reshape(R * N, 16),
                          ak1.reshape(R * N, 16), _mvec(pm1), z16)
    den1 = _add2(dp1)
    op1 = _msg_kernel(et, src, dst, ex1, den1, xr1.reshape(R * N, HID), z128)

    xr2, aq2, ak2, pm2 = _tables_mid(op1, bias1, w_rel2, q2, k2)
    ex2, dp2 = _ab_kernel(et, src, dst, aq2.reshape(R * N, 16),
                          ak2.reshape(R * N, 16), _mvec(pm2), z16)
    den2 = _add2(dp2)
    op2 = _msg_kernel(et, src, dst, ex2, den2, xr2.reshape(R * N, HID), z128)

    return _final(op2, bias2, skip)


# speedup vs baseline: 13.6265x; 1.1276x over previous
import functools

import jax
import jax.numpy as jnp
from jax import lax
from jax.experimental import pallas as pl
from jax.experimental.pallas import tpu as pltpu
from jax.experimental.pallas import tpu_sc as plsc

N = 10000
E = 320000
R = 8
H = 4
KG_DIM = 128
IN_CH = 256
HID = 128
OUT = 128

NC = 2
NS = 16
NW = NC * NS
CH = 128
ER = E // CH
NCHUNK1 = E // CH
NFULL1 = NCHUNK1 // NW
XTRA1 = NCHUNK1 - NFULL1 * NW
NP = 10240
RPS = NP // NS

_B = 1000
_NB = N // _B

f32 = jnp.float32
i32 = jnp.int32


def _leaky(x, s):
    return jnp.where(x >= 0, x, s * x)


def _prep_body(kg_ref, ccle_ref, w1_ref, b1_ref, w2_ref, b2_ref,
               sw1_ref, sb1_ref, sw2_ref, sb2_ref, xin_ref, skip_ref):
    h = jnp.dot(ccle_ref[...], w1_ref[...], preferred_element_type=f32) + b1_ref[...]
    h = _leaky(h, 0.01)
    co = jnp.dot(h, w2_ref[...], preferred_element_type=f32) + b2_ref[...]
    kg = kg_ref[...]
    xin_ref[:, :KG_DIM] = kg
    xin_ref[:, KG_DIM:] = co
    xi = jnp.concatenate([kg, co], axis=1)
    s = jnp.dot(xi, sw1_ref[...], preferred_element_type=f32) + sb1_ref[...]
    s = _leaky(s, 0.01)
    skip_ref[...] = jnp.dot(s, sw2_ref[...], preferred_element_type=f32) + sb2_ref[...]


def _prep(kg, ccle, w1, b1, w2, b2, sw1, sb1, sw2, sb2):
    full = lambda shape: pl.BlockSpec(shape, lambda j: tuple(0 for _ in shape))
    return pl.pallas_call(
        _prep_body,
        grid=(_NB,),
        in_specs=[
            pl.BlockSpec((_B, KG_DIM), lambda j: (j, 0)),
            pl.BlockSpec((_B, 4), lambda j: (j, 0)),
            full((4, 32)), full((32,)), full((32, 128)), full((128,)),
            full((IN_CH, HID)), full((HID,)), full((HID, OUT)), full((OUT,)),
        ],
        out_specs=[
            pl.BlockSpec((_B, IN_CH), lambda j: (j, 0)),
            pl.BlockSpec((_B, OUT), lambda j: (j, 0)),
        ],
        out_shape=[
            jax.ShapeDtypeStruct((N, IN_CH), f32),
            jax.ShapeDtypeStruct((N, OUT), f32),
        ],
    )(kg, ccle, w1, b1, w2, b2, sw1, sb1, sw2, sb2)


def _emit_tables(xb, q_ref, k_ref, xr_ref, aq_ref, ak_ref, pm_ref):
    xr_ref[0] = xb
    aq = jnp.dot(xb, q_ref[...], preferred_element_type=f32)
    ak = jnp.dot(xb, k_ref[...], preferred_element_type=f32)
    pad = jnp.zeros((xb.shape[0], 16 - H), f32)
    aq_ref[0] = jnp.concatenate([aq, pad], axis=1)
    ak_ref[0] = jnp.concatenate([ak, pad], axis=1)
    pm_ref[...] = jnp.concatenate(
        [jnp.max(aq, axis=0), jnp.max(ak, axis=0)]).reshape(1, 1, 2 * H)


def _tables_body(x_ref, wr_ref, q_ref, k_ref, xr_ref, aq_ref, ak_ref, pm_ref):
    xb = jnp.dot(x_ref[...], wr_ref[0], preferred_element_type=f32)
    _emit_tables(xb, q_ref, k_ref, xr_ref, aq_ref, ak_ref, pm_ref)


def _tables_mid_body(p_ref, b_ref, wr_ref, q_ref, k_ref,
                     xr_ref, aq_ref, ak_ref, pm_ref):
    x = _leaky(p_ref[0] + p_ref[1] + b_ref[...], 0.01)
    xb = jnp.dot(x, wr_ref[0], preferred_element_type=f32)
    _emit_tables(xb, q_ref, k_ref, xr_ref, aq_ref, ak_ref, pm_ref)


def _table_out_specs():
    return dict(
        out_specs=[
            pl.BlockSpec((1, _B, HID), lambda r, j: (r, j, 0)),
            pl.BlockSpec((1, _B, 16), lambda r, j: (r, j, 0)),
            pl.BlockSpec((1, _B, 16), lambda r, j: (r, j, 0)),
            pl.BlockSpec((1, 1, 2 * H), lambda r, j: (r * _NB + j, 0, 0)),
        ],
        out_shape=[
            jax.ShapeDtypeStruct((R, N, HID), f32),
            jax.ShapeDtypeStruct((R, N, 16), f32),
            jax.ShapeDtypeStruct((R, N, 16), f32),
            jax.ShapeDtypeStruct((R * _NB, 1, 2 * H), f32),
        ],
    )


def _tables(x, wr, q, k):
    in_ch = x.shape[1]
    return pl.pallas_call(
        _tables_body,
        grid=(R, _NB),
        in_specs=[
            pl.BlockSpec((_B, in_ch), lambda r, j: (j, 0)),
            pl.BlockSpec((1, in_ch, HID), lambda r, j: (r, 0, 0)),
            pl.BlockSpec((HID, H), lambda r, j: (0, 0)),
            pl.BlockSpec((HID, H), lambda r, j: (0, 0)),
        ],
        **_table_out_specs(),
    )(x, wr, q, k)


def _tables_mid(p, b, wr, q, k):
    return pl.pallas_call(
        _tables_mid_body,
        grid=(R, _NB),
        in_specs=[
            pl.BlockSpec((NC, _B, HID), lambda r, j: (0, j, 0)),
            pl.BlockSpec((HID,), lambda r, j: (0,)),
            pl.BlockSpec((1, HID, HID), lambda r, j: (r, 0, 0)),
            pl.BlockSpec((HID, H), lambda r, j: (0, 0)),
            pl.BlockSpec((HID, H), lambda r, j: (0, 0)),
        ],
        **_table_out_specs(),
    )(p, b, wr, q, k)


def _mvec(pm):
    mx = jnp.max(pm.reshape(-1, 2 * H), axis=0)
    m4 = _leaky(mx[:H] + mx[H:], 0.2)
    return jnp.concatenate([m4, jnp.zeros((16 - H,), f32)])


def _add2_body(p_ref, o_ref):
    o_ref[...] = p_ref[0] + p_ref[1]


def _add2(p):
    bp = NP // 10
    return pl.pallas_call(
        _add2_body,
        grid=(10,),
        in_specs=[pl.BlockSpec((NC, bp, 16), lambda j: (0, j, 0))],
        out_specs=pl.BlockSpec((bp, 16), lambda j: (j, 0)),
        out_shape=jax.ShapeDtypeStruct((NP, 16), f32),
    )(p)


def _pack_body(et_ref, src_ref, dst_ref, o_ref):
    e = et_ref[...] * N
    o_ref[:, 0, :] = e + dst_ref[...]
    o_ref[:, 1, :] = e + src_ref[...]
    o_ref[:, 2, :] = dst_ref[...]


def _pack(et, srcv, dstv):
    return pl.pallas_call(
        _pack_body,
        grid=(1,),
        in_specs=[pl.BlockSpec((ER, CH), lambda j: (0, 0)) for _ in range(3)],
        out_specs=pl.BlockSpec((ER, 3, CH), lambda j: (0, 0, 0)),
        out_shape=jax.ShapeDtypeStruct((ER, 3, CH), i32),
    )(et, srcv, dstv)


def _final_body(p_ref, b_ref, skip_ref, o_ref):
    o_ref[...] = _leaky(p_ref[0] + p_ref[1] + b_ref[...] + skip_ref[...], 0.01)


def _final(p, b, skip):
    return pl.pallas_call(
        _final_body,
        grid=(_NB,),
        in_specs=[
            pl.BlockSpec((NC, _B, OUT), lambda j: (0, j, 0)),
            pl.BlockSpec((OUT,), lambda j: (0,)),
            pl.BlockSpec((_B, OUT), lambda j: (j, 0)),
        ],
        out_specs=pl.BlockSpec((_B, OUT), lambda j: (j, 0)),
        out_shape=jax.ShapeDtypeStruct((N, OUT), f32),
    )(p, b, skip)


_MESH = plsc.VectorSubcoreMesh(core_axis_name="c", subcore_axis_name="s")
_SC_PARAMS = pltpu.CompilerParams(use_tc_tiling_on_sc=False)


def _worker_id():
    return lax.axis_index("s") * NC + lax.axis_index("c")


def _nchunks(wid):
    return jnp.where(wid < XTRA1, NFULL1 + 1, NFULL1).astype(i32)


@functools.partial(
    pl.kernel,
    out_type=(
        jax.ShapeDtypeStruct((E, 16), f32),
        jax.ShapeDtypeStruct((NC, NP, 16), f32),
    ),
    mesh=_MESH,
    scratch_types=(
        pltpu.VMEM((3, CH), i32),
        pltpu.VMEM((CH, 16), f32),
        pltpu.VMEM((CH, 16), f32),
        pltpu.VMEM((CH, 16), f32),
        pltpu.VMEM((16,), f32),
        pltpu.VMEM_SHARED((NP, 16), f32),
    ),
    compiler_params=_SC_PARAMS,
)
def _ab_kernel(pk_h, aq_h, ak_h, m_h, z16_h, ex_h, dp_h,
               pk_v, ai, aj, ex_v, m_v, acc_s):
    cid = lax.axis_index("c")
    sid = lax.axis_index("s")
    wid = _worker_id()
    pltpu.sync_copy(m_h, m_v)
    pltpu.sync_copy(z16_h.at[pl.ds(sid * RPS, RPS), :],
                    acc_s.at[pl.ds(sid * RPS, RPS), :])
    plsc.subcore_barrier()
    mv = m_v[...]

    def body(i, carry):
        ch = wid + i * NW
        base = ch * CH
        pltpu.sync_copy(pk_h.at[ch], pk_v)
        pltpu.sync_copy(aq_h.at[pk_v.at[0]], ai)
        pltpu.sync_copy(ak_h.at[pk_v.at[1]], aj)

        def erow(r, c):
            a = ai[r, :] + aj[r, :]
            a = jnp.where(a >= 0, a, 0.2 * a)
            ex_v[r, :] = jnp.exp(a - mv)
            return c
        lax.fori_loop(0, CH, erow, 0, unroll=4)
        pltpu.sync_copy(ex_v, ex_h.at[pl.ds(base, CH), :])
        pltpu.sync_copy(ex_v, acc_s.at[pk_v.at[2]], add=True)
        return carry

    lax.fori_loop(0, _nchunks(wid), body, 0)
    plsc.subcore_barrier()
    pltpu.sync_copy(acc_s.at[pl.ds(sid * RPS, RPS), :],
                    dp_h.at[cid, pl.ds(sid * RPS, RPS), :])


@functools.partial(
    pl.kernel,
    out_type=jax.ShapeDtypeStruct((NC, NP, HID), f32),
    mesh=_MESH,
    scratch_types=(
        pltpu.VMEM((3, CH), i32),
        pltpu.VMEM((CH, 16), f32),
        pltpu.VMEM((CH, 16), f32),
        pltpu.VMEM((CH, HID), f32),
        pltpu.VMEM_SHARED((NP, HID), f32),
    ),
    compiler_params=_SC_PARAMS,
)
def _msg_kernel(pk_h, exb_h, den_h, xr_h, z128_h, op_h,
                pk_v, ex_v, den_v, xr_v, acc_s):
    cid = lax.axis_index("c")
    sid = lax.axis_index("s")
    wid = _worker_id()
    pltpu.sync_copy(z128_h.at[pl.ds(sid * RPS, RPS), :],
                    acc_s.at[pl.ds(sid * RPS, RPS), :])
    plsc.subcore_barrier()

    def body(i, carry):
        ch = wid + i * NW
        base = ch * CH
        pltpu.sync_copy(pk_h.at[ch], pk_v)
        pltpu.sync_copy(exb_h.at[pl.ds(base, CH), :], ex_v)
        pltpu.sync_copy(den_h.at[pk_v.at[2]], den_v)
        pltpu.sync_copy(xr_h.at[pk_v.at[1]], xr_v)

        def erow(r, c):
            at = ex_v[r, :] / (den_v[r, :] + 1e-16)
            for v in range(HID // 16):
                sl = pl.ds(v * 16, 16)
                xr_v[r, sl] = xr_v[r, sl] * at[v * 16 // 32]
            return c
        lax.fori_loop(0, CH, erow, 0, unroll=2)
        pltpu.sync_copy(xr_v, acc_s.at[pk_v.at[2]], add=True)
        return carry

    lax.fori_loop(0, _nchunks(wid), body, 0)
    plsc.subcore_barrier()
    pltpu.sync_copy(acc_s.at[pl.ds(sid * RPS, RPS), :],
                    op_h.at[cid, pl.ds(sid * RPS, RPS), :])


def kernel(kg_emb, ccle, node_id, edge_index, edge_type,
           ccle_w1, ccle_b1, ccle_w2, ccle_b2,
           w_rel1, q1, k1, bias1, w_rel2, q2, k2, bias2,
           skip_w1, skip_b1, skip_w2, skip_b2):
    srcv = edge_index[0].reshape(ER, CH)
    dstv = edge_index[1].reshape(ER, CH)
    etv = edge_type.reshape(ER, CH)
    z16 = jnp.zeros((NP, 16), f32)
    z128 = jnp.zeros((NP, HID), f32)

    xin, skip = _prep(kg_emb, ccle, ccle_w1, ccle_b1, ccle_w2, ccle_b2,
                      skip_w1, skip_b1, skip_w2, skip_b2)
    pk = _pack(etv, srcv, dstv)

    xr1, aq1, ak1, pm1 = _tables(xin, w_rel1, q1, k1)
    ex1, dp1 = _ab_kernel(pk, aq1.reshape(R * N, 16),
                          ak1.reshape(R * N, 16), _mvec(pm1), z16)
    den1 = _add2(dp1)
    op1 = _msg_kernel(pk, ex1, den1, xr1.reshape(R * N, HID), z128)

    xr2, aq2, ak2, pm2 = _tables_mid(op1, bias1, w_rel2, q2, k2)
    ex2, dp2 = _ab_kernel(pk, aq2.reshape(R * N, 16),
                          ak2.reshape(R * N, 16), _mvec(pm2), z16)
    den2 = _add2(dp2)
    op2 = _msg_kernel(pk, ex2, den2, xr2.reshape(R * N, HID), z128)

    return _final(op2, bias2, skip)


# speedup vs baseline: 15.6808x; 1.1508x over previous
import functools

import jax
import jax.numpy as jnp
from jax import lax
from jax.experimental import pallas as pl
from jax.experimental.pallas import tpu as pltpu
from jax.experimental.pallas import tpu_sc as plsc

N = 10000
E = 320000
R = 8
H = 4
KG_DIM = 128
IN_CH = 256
HID = 128
OUT = 128

NC = 2
NS = 16
NW = NC * NS
CH = 128
ER = E // CH
NCHUNK1 = E // CH
NFULL1 = NCHUNK1 // NW
XTRA1 = NCHUNK1 - NFULL1 * NW
NP = 10240
RPS = NP // NS

_B = 1000
_NB = N // _B

f32 = jnp.float32
i32 = jnp.int32


def _leaky(x, s):
    return jnp.where(x >= 0, x, s * x)


def _prep_body(kg_ref, ccle_ref, w1_ref, b1_ref, w2_ref, b2_ref,
               sw1_ref, sb1_ref, sw2_ref, sb2_ref, xin_ref, skip_ref):
    h = jnp.dot(ccle_ref[...], w1_ref[...], preferred_element_type=f32) + b1_ref[...]
    h = _leaky(h, 0.01)
    co = jnp.dot(h, w2_ref[...], preferred_element_type=f32) + b2_ref[...]
    kg = kg_ref[...]
    xin_ref[:, :KG_DIM] = kg
    xin_ref[:, KG_DIM:] = co
    xi = jnp.concatenate([kg, co], axis=1)
    s = jnp.dot(xi, sw1_ref[...], preferred_element_type=f32) + sb1_ref[...]
    s = _leaky(s, 0.01)
    skip_ref[...] = jnp.dot(s, sw2_ref[...], preferred_element_type=f32) + sb2_ref[...]


def _prep(kg, ccle, w1, b1, w2, b2, sw1, sb1, sw2, sb2):
    full = lambda shape: pl.BlockSpec(shape, lambda j: tuple(0 for _ in shape))
    return pl.pallas_call(
        _prep_body,
        grid=(_NB,),
        in_specs=[
            pl.BlockSpec((_B, KG_DIM), lambda j: (j, 0)),
            pl.BlockSpec((_B, 4), lambda j: (j, 0)),
            full((4, 32)), full((32,)), full((32, 128)), full((128,)),
            full((IN_CH, HID)), full((HID,)), full((HID, OUT)), full((OUT,)),
        ],
        out_specs=[
            pl.BlockSpec((_B, IN_CH), lambda j: (j, 0)),
            pl.BlockSpec((_B, OUT), lambda j: (j, 0)),
        ],
        out_shape=[
            jax.ShapeDtypeStruct((N, IN_CH), f32),
            jax.ShapeDtypeStruct((N, OUT), f32),
        ],
    )(kg, ccle, w1, b1, w2, b2, sw1, sb1, sw2, sb2)


def _emit_tables(xb, q_ref, k_ref, xr_ref, aq_ref, ak_ref, pm_ref):
    xr_ref[0] = xb
    aq = jnp.dot(xb, q_ref[...], preferred_element_type=f32)
    ak = jnp.dot(xb, k_ref[...], preferred_element_type=f32)
    pad = jnp.zeros((xb.shape[0], 16 - H), f32)
    aq_ref[0] = jnp.concatenate([aq, pad], axis=1)
    ak_ref[0] = jnp.concatenate([ak, pad], axis=1)
    pm_ref[...] = jnp.concatenate(
        [jnp.max(aq, axis=0), jnp.max(ak, axis=0)]).reshape(1, 1, 2 * H)


def _tables_body(x_ref, wr_ref, q_ref, k_ref, xr_ref, aq_ref, ak_ref, pm_ref):
    xb = jnp.dot(x_ref[...], wr_ref[0], preferred_element_type=f32)
    _emit_tables(xb, q_ref, k_ref, xr_ref, aq_ref, ak_ref, pm_ref)


def _tables_mid_body(p_ref, b_ref, wr_ref, q_ref, k_ref,
                     xr_ref, aq_ref, ak_ref, pm_ref):
    x = _leaky(p_ref[0] + p_ref[1] + b_ref[...], 0.01)
    xb = jnp.dot(x, wr_ref[0], preferred_element_type=f32)
    _emit_tables(xb, q_ref, k_ref, xr_ref, aq_ref, ak_ref, pm_ref)


def _table_out_specs():
    return dict(
        out_specs=[
            pl.BlockSpec((1, _B, HID), lambda r, j: (r, j, 0)),
            pl.BlockSpec((1, _B, 16), lambda r, j: (r, j, 0)),
            pl.BlockSpec((1, _B, 16), lambda r, j: (r, j, 0)),
            pl.BlockSpec((1, 1, 2 * H), lambda r, j: (r * _NB + j, 0, 0)),
        ],
        out_shape=[
            jax.ShapeDtypeStruct((R, N, HID), f32),
            jax.ShapeDtypeStruct((R, N, 16), f32),
            jax.ShapeDtypeStruct((R, N, 16), f32),
            jax.ShapeDtypeStruct((R * _NB, 1, 2 * H), f32),
        ],
    )


def _tables(x, wr, q, k):
    in_ch = x.shape[1]
    return pl.pallas_call(
        _tables_body,
        grid=(R, _NB),
        in_specs=[
            pl.BlockSpec((_B, in_ch), lambda r, j: (j, 0)),
            pl.BlockSpec((1, in_ch, HID), lambda r, j: (r, 0, 0)),
            pl.BlockSpec((HID, H), lambda r, j: (0, 0)),
            pl.BlockSpec((HID, H), lambda r, j: (0, 0)),
        ],
        **_table_out_specs(),
    )(x, wr, q, k)


def _tables_mid(p, b, wr, q, k):
    return pl.pallas_call(
        _tables_mid_body,
        grid=(R, _NB),
        in_specs=[
            pl.BlockSpec((NC, _B, HID), lambda r, j: (0, j, 0)),
            pl.BlockSpec((HID,), lambda r, j: (0,)),
            pl.BlockSpec((1, HID, HID), lambda r, j: (r, 0, 0)),
            pl.BlockSpec((HID, H), lambda r, j: (0, 0)),
            pl.BlockSpec((HID, H), lambda r, j: (0, 0)),
        ],
        **_table_out_specs(),
    )(p, b, wr, q, k)


def _mvec(pm):
    mx = jnp.max(pm.reshape(-1, 2 * H), axis=0)
    m4 = _leaky(mx[:H] + mx[H:], 0.2)
    return jnp.concatenate([m4, jnp.zeros((16 - H,), f32)])


def _add2_body(p_ref, o_ref):
    o_ref[...] = p_ref[0] + p_ref[1]


def _add2(p):
    bp = NP // 10
    return pl.pallas_call(
        _add2_body,
        grid=(10,),
        in_specs=[pl.BlockSpec((NC, bp, 16), lambda j: (0, j, 0))],
        out_specs=pl.BlockSpec((bp, 16), lambda j: (j, 0)),
        out_shape=jax.ShapeDtypeStruct((NP, 16), f32),
    )(p)


def _pack_body(et_ref, src_ref, dst_ref, o_ref):
    e = et_ref[...] * N
    o_ref[:, 0, :] = e + dst_ref[...]
    o_ref[:, 1, :] = e + src_ref[...]
    o_ref[:, 2, :] = dst_ref[...]


def _pack(et, srcv, dstv):
    return pl.pallas_call(
        _pack_body,
        grid=(1,),
        in_specs=[pl.BlockSpec((ER, CH), lambda j: (0, 0)) for _ in range(3)],
        out_specs=pl.BlockSpec((ER, 3, CH), lambda j: (0, 0, 0)),
        out_shape=jax.ShapeDtypeStruct((ER, 3, CH), i32),
    )(et, srcv, dstv)


def _final_body(p_ref, b_ref, skip_ref, o_ref):
    o_ref[...] = _leaky(p_ref[0] + p_ref[1] + b_ref[...] + skip_ref[...], 0.01)


def _final(p, b, skip):
    return pl.pallas_call(
        _final_body,
        grid=(_NB,),
        in_specs=[
            pl.BlockSpec((NC, _B, OUT), lambda j: (0, j, 0)),
            pl.BlockSpec((OUT,), lambda j: (0,)),
            pl.BlockSpec((_B, OUT), lambda j: (j, 0)),
        ],
        out_specs=pl.BlockSpec((_B, OUT), lambda j: (j, 0)),
        out_shape=jax.ShapeDtypeStruct((N, OUT), f32),
    )(p, b, skip)


_MESH = plsc.VectorSubcoreMesh(core_axis_name="c", subcore_axis_name="s")
_SC_PARAMS = pltpu.CompilerParams(use_tc_tiling_on_sc=False)


def _worker_id():
    return lax.axis_index("s") * NC + lax.axis_index("c")


def _nchunks(wid):
    return jnp.where(wid < XTRA1, NFULL1 + 1, NFULL1).astype(i32)


@functools.partial(
    pl.kernel,
    out_type=(
        jax.ShapeDtypeStruct((E, 16), f32),
        jax.ShapeDtypeStruct((NC, NP, 16), f32),
    ),
    mesh=_MESH,
    scratch_types=(
        pltpu.VMEM((3, CH), i32),
        pltpu.VMEM((CH, 16), f32),
        pltpu.VMEM((CH, 16), f32),
        pltpu.VMEM((CH, 16), f32),
        pltpu.VMEM((16,), f32),
        pltpu.VMEM_SHARED((NP, 16), f32),
        pltpu.SemaphoreType.DMA,
    ),
    compiler_params=_SC_PARAMS,
)
def _ab_kernel(pk_h, aq_h, ak_h, m_h, z16_h, ex_h, dp_h,
               pk_v, ai, aj, ex_v, m_v, acc_s, sem):
    cid = lax.axis_index("c")
    sid = lax.axis_index("s")
    wid = _worker_id()
    pltpu.sync_copy(m_h, m_v)
    pltpu.sync_copy(z16_h.at[pl.ds(sid * RPS, RPS), :],
                    acc_s.at[pl.ds(sid * RPS, RPS), :])
    plsc.subcore_barrier()
    mv = m_v[...]

    def body(i, carry):
        ch = wid + i * NW
        base = ch * CH
        pltpu.sync_copy(pk_h.at[ch], pk_v)
        d1 = pltpu.async_copy(aq_h.at[pk_v.at[0]], ai, sem)
        d2 = pltpu.async_copy(ak_h.at[pk_v.at[1]], aj, sem)
        d1.wait()
        d2.wait()

        def erow(r, c):
            a = ai[r, :] + aj[r, :]
            a = jnp.where(a >= 0, a, 0.2 * a)
            ex_v[r, :] = jnp.exp(a - mv)
            return c
        lax.fori_loop(0, CH, erow, 0, unroll=4)
        pltpu.sync_copy(ex_v, ex_h.at[pl.ds(base, CH), :])
        pltpu.sync_copy(ex_v, acc_s.at[pk_v.at[2]], add=True)
        return carry

    lax.fori_loop(0, _nchunks(wid), body, 0)
    plsc.subcore_barrier()
    pltpu.sync_copy(acc_s.at[pl.ds(sid * RPS, RPS), :],
                    dp_h.at[cid, pl.ds(sid * RPS, RPS), :])


@functools.partial(
    pl.kernel,
    out_type=jax.ShapeDtypeStruct((NC, NP, HID), f32),
    mesh=_MESH,
    scratch_types=(
        pltpu.VMEM((3, CH), i32),
        pltpu.VMEM((CH, 16), f32),
        pltpu.VMEM((CH, 16), f32),
        pltpu.VMEM((CH, HID), f32),
        pltpu.VMEM_SHARED((NP, HID), f32),
        pltpu.SemaphoreType.DMA,
    ),
    compiler_params=_SC_PARAMS,
)
def _msg_kernel(pk_h, exb_h, den_h, xr_h, z128_h, op_h,
                pk_v, ex_v, den_v, xr_v, acc_s, sem):
    cid = lax.axis_index("c")
    sid = lax.axis_index("s")
    wid = _worker_id()
    pltpu.sync_copy(z128_h.at[pl.ds(sid * RPS, RPS), :],
                    acc_s.at[pl.ds(sid * RPS, RPS), :])
    plsc.subcore_barrier()

    def body(i, carry):
        ch = wid + i * NW
        base = ch * CH
        pltpu.sync_copy(pk_h.at[ch], pk_v)
        d1 = pltpu.async_copy(exb_h.at[pl.ds(base, CH), :], ex_v, sem)
        d2 = pltpu.async_copy(den_h.at[pk_v.at[2]], den_v, sem)
        d3 = pltpu.async_copy(xr_h.at[pk_v.at[1]], xr_v, sem)
        d1.wait()
        d2.wait()
        d3.wait()

        def erow(r, c):
            at = ex_v[r, :] / (den_v[r, :] + 1e-16)
            for v in range(HID // 16):
                sl = pl.ds(v * 16, 16)
                xr_v[r, sl] = xr_v[r, sl] * at[v * 16 // 32]
            return c
        lax.fori_loop(0, CH, erow, 0, unroll=2)
        pltpu.sync_copy(xr_v, acc_s.at[pk_v.at[2]], add=True)
        return carry

    lax.fori_loop(0, _nchunks(wid), body, 0)
    plsc.subcore_barrier()
    pltpu.sync_copy(acc_s.at[pl.ds(sid * RPS, RPS), :],
                    op_h.at[cid, pl.ds(sid * RPS, RPS), :])


def kernel(kg_emb, ccle, node_id, edge_index, edge_type,
           ccle_w1, ccle_b1, ccle_w2, ccle_b2,
           w_rel1, q1, k1, bias1, w_rel2, q2, k2, bias2,
           skip_w1, skip_b1, skip_w2, skip_b2):
    srcv = edge_index[0].reshape(ER, CH)
    dstv = edge_index[1].reshape(ER, CH)
    etv = edge_type.reshape(ER, CH)
    z16 = jnp.zeros((NP, 16), f32)
    z128 = jnp.zeros((NP, HID), f32)

    xin, skip = _prep(kg_emb, ccle, ccle_w1, ccle_b1, ccle_w2, ccle_b2,
                      skip_w1, skip_b1, skip_w2, skip_b2)
    pk = _pack(etv, srcv, dstv)

    xr1, aq1, ak1, pm1 = _tables(xin, w_rel1, q1, k1)
    ex1, dp1 = _ab_kernel(pk, aq1.reshape(R * N, 16),
                          ak1.reshape(R * N, 16), _mvec(pm1), z16)
    den1 = _add2(dp1)
    op1 = _msg_kernel(pk, ex1, den1, xr1.reshape(R * N, HID), z128)

    xr2, aq2, ak2, pm2 = _tables_mid(op1, bias1, w_rel2, q2, k2)
    ex2, dp2 = _ab_kernel(pk, aq2.reshape(R * N, 16),
                          ak2.reshape(R * N, 16), _mvec(pm2), z16)
    den2 = _add2(dp2)
    op2 = _msg_kernel(pk, ex2, den2, xr2.reshape(R * N, HID), z128)

    return _final(op2, bias2, skip)


# speedup vs baseline: 16.0363x; 1.0227x over previous
import functools

import jax
import jax.numpy as jnp
from jax import lax
from jax.experimental import pallas as pl
from jax.experimental.pallas import tpu as pltpu
from jax.experimental.pallas import tpu_sc as plsc

N = 10000
E = 320000
R = 8
H = 4
KG_DIM = 128
IN_CH = 256
HID = 128
OUT = 128

NC = 2
NS = 16
NW = NC * NS
CH = 128
ER = E // CH
NCHUNK1 = E // CH
NPAIR = E // (2 * CH)
NFP = NPAIR // NW
XTP = NPAIR - NFP * NW
NFULL1 = NCHUNK1 // NW
XTRA1 = NCHUNK1 - NFULL1 * NW
NP = 10240
RPS = NP // NS

_B = 1000
_NB = N // _B

f32 = jnp.float32
i32 = jnp.int32


def _leaky(x, s):
    return jnp.where(x >= 0, x, s * x)


def _prep_body(kg_ref, ccle_ref, w1_ref, b1_ref, w2_ref, b2_ref,
               sw1_ref, sb1_ref, sw2_ref, sb2_ref, xin_ref, skip_ref):
    h = jnp.dot(ccle_ref[...], w1_ref[...], preferred_element_type=f32) + b1_ref[...]
    h = _leaky(h, 0.01)
    co = jnp.dot(h, w2_ref[...], preferred_element_type=f32) + b2_ref[...]
    kg = kg_ref[...]
    xin_ref[:, :KG_DIM] = kg
    xin_ref[:, KG_DIM:] = co
    xi = jnp.concatenate([kg, co], axis=1)
    s = jnp.dot(xi, sw1_ref[...], preferred_element_type=f32) + sb1_ref[...]
    s = _leaky(s, 0.01)
    skip_ref[...] = jnp.dot(s, sw2_ref[...], preferred_element_type=f32) + sb2_ref[...]


def _prep(kg, ccle, w1, b1, w2, b2, sw1, sb1, sw2, sb2):
    full = lambda shape: pl.BlockSpec(shape, lambda j: tuple(0 for _ in shape))
    return pl.pallas_call(
        _prep_body,
        grid=(_NB,),
        in_specs=[
            pl.BlockSpec((_B, KG_DIM), lambda j: (j, 0)),
            pl.BlockSpec((_B, 4), lambda j: (j, 0)),
            full((4, 32)), full((32,)), full((32, 128)), full((128,)),
            full((IN_CH, HID)), full((HID,)), full((HID, OUT)), full((OUT,)),
        ],
        out_specs=[
            pl.BlockSpec((_B, IN_CH), lambda j: (j, 0)),
            pl.BlockSpec((_B, OUT), lambda j: (j, 0)),
        ],
        out_shape=[
            jax.ShapeDtypeStruct((N, IN_CH), f32),
            jax.ShapeDtypeStruct((N, OUT), f32),
        ],
    )(kg, ccle, w1, b1, w2, b2, sw1, sb1, sw2, sb2)


def _emit_tables(xb, q_ref, k_ref, xr_ref, aq_ref, ak_ref, pm_ref):
    xr_ref[0] = xb
    aq = jnp.dot(xb, q_ref[...], preferred_element_type=f32)
    ak = jnp.dot(xb, k_ref[...], preferred_element_type=f32)
    pad = jnp.zeros((xb.shape[0], 16 - H), f32)
    aq_ref[0] = jnp.concatenate([aq, pad], axis=1)
    ak_ref[0] = jnp.concatenate([ak, pad], axis=1)
    pm_ref[...] = jnp.concatenate(
        [jnp.max(aq, axis=0), jnp.max(ak, axis=0)]).reshape(1, 1, 2 * H)


def _tables_body(x_ref, wr_ref, q_ref, k_ref, xr_ref, aq_ref, ak_ref, pm_ref):
    xb = jnp.dot(x_ref[...], wr_ref[0], preferred_element_type=f32)
    _emit_tables(xb, q_ref, k_ref, xr_ref, aq_ref, ak_ref, pm_ref)


def _tables_mid_body(p_ref, b_ref, wr_ref, q_ref, k_ref,
                     xr_ref, aq_ref, ak_ref, pm_ref):
    x = _leaky(p_ref[0] + p_ref[1] + b_ref[...], 0.01)
    xb = jnp.dot(x, wr_ref[0], preferred_element_type=f32)
    _emit_tables(xb, q_ref, k_ref, xr_ref, aq_ref, ak_ref, pm_ref)


def _table_out_specs():
    return dict(
        out_specs=[
            pl.BlockSpec((1, _B, HID), lambda r, j: (r, j, 0)),
            pl.BlockSpec((1, _B, 16), lambda r, j: (r, j, 0)),
            pl.BlockSpec((1, _B, 16), lambda r, j: (r, j, 0)),
            pl.BlockSpec((1, 1, 2 * H), lambda r, j: (r * _NB + j, 0, 0)),
        ],
        out_shape=[
            jax.ShapeDtypeStruct((R, N, HID), f32),
            jax.ShapeDtypeStruct((R, N, 16), f32),
            jax.ShapeDtypeStruct((R, N, 16), f32),
            jax.ShapeDtypeStruct((R * _NB, 1, 2 * H), f32),
        ],
    )


def _tables(x, wr, q, k):
    in_ch = x.shape[1]
    return pl.pallas_call(
        _tables_body,
        grid=(R, _NB),
        in_specs=[
            pl.BlockSpec((_B, in_ch), lambda r, j: (j, 0)),
            pl.BlockSpec((1, in_ch, HID), lambda r, j: (r, 0, 0)),
            pl.BlockSpec((HID, H), lambda r, j: (0, 0)),
            pl.BlockSpec((HID, H), lambda r, j: (0, 0)),
        ],
        **_table_out_specs(),
    )(x, wr, q, k)


def _tables_mid(p, b, wr, q, k):
    return pl.pallas_call(
        _tables_mid_body,
        grid=(R, _NB),
        in_specs=[
            pl.BlockSpec((NC, _B, HID), lambda r, j: (0, j, 0)),
            pl.BlockSpec((HID,), lambda r, j: (0,)),
            pl.BlockSpec((1, HID, HID), lambda r, j: (r, 0, 0)),
            pl.BlockSpec((HID, H), lambda r, j: (0, 0)),
            pl.BlockSpec((HID, H), lambda r, j: (0, 0)),
        ],
        **_table_out_specs(),
    )(p, b, wr, q, k)


def _mvec(pm):
    mx = jnp.max(pm.reshape(-1, 2 * H), axis=0)
    m4 = _leaky(mx[:H] + mx[H:], 0.2)
    return jnp.concatenate([m4, jnp.zeros((16 - H,), f32)])


def _add2_body(p_ref, o_ref):
    o_ref[...] = p_ref[0] + p_ref[1]


def _add2(p):
    bp = NP // 10
    return pl.pallas_call(
        _add2_body,
        grid=(10,),
        in_specs=[pl.BlockSpec((NC, bp, 16), lambda j: (0, j, 0))],
        out_specs=pl.BlockSpec((bp, 16), lambda j: (j, 0)),
        out_shape=jax.ShapeDtypeStruct((NP, 16), f32),
    )(p)


def _pack_body(et_ref, src_ref, dst_ref, o_ref):
    e = et_ref[...] * N
    o_ref[:, 0, :] = e + dst_ref[...]
    o_ref[:, 1, :] = e + src_ref[...]
    o_ref[:, 2, :] = dst_ref[...]


def _pack(et, srcv, dstv):
    return pl.pallas_call(
        _pack_body,
        grid=(1,),
        in_specs=[pl.BlockSpec((ER, CH), lambda j: (0, 0)) for _ in range(3)],
        out_specs=pl.BlockSpec((ER, 3, CH), lambda j: (0, 0, 0)),
        out_shape=jax.ShapeDtypeStruct((ER, 3, CH), i32),
    )(et, srcv, dstv)


def _final_body(p_ref, b_ref, skip_ref, o_ref):
    o_ref[...] = _leaky(p_ref[0] + p_ref[1] + b_ref[...] + skip_ref[...], 0.01)


def _final(p, b, skip):
    return pl.pallas_call(
        _final_body,
        grid=(_NB,),
        in_specs=[
            pl.BlockSpec((NC, _B, OUT), lambda j: (0, j, 0)),
            pl.BlockSpec((OUT,), lambda j: (0,)),
            pl.BlockSpec((_B, OUT), lambda j: (j, 0)),
        ],
        out_specs=pl.BlockSpec((_B, OUT), lambda j: (j, 0)),
        out_shape=jax.ShapeDtypeStruct((N, OUT), f32),
    )(p, b, skip)


_MESH = plsc.VectorSubcoreMesh(core_axis_name="c", subcore_axis_name="s")
_SC_PARAMS = pltpu.CompilerParams(use_tc_tiling_on_sc=False)


def _worker_id():
    return lax.axis_index("s") * NC + lax.axis_index("c")


def _nchunks(wid):
    return jnp.where(wid < XTRA1, NFULL1 + 1, NFULL1).astype(i32)


def _npairs(wid):
    return jnp.where(wid < XTP, NFP + 1, NFP).astype(i32)


@functools.partial(
    pl.kernel,
    out_type=(
        jax.ShapeDtypeStruct((E, 16), f32),
        jax.ShapeDtypeStruct((NC, NP, 16), f32),
    ),
    mesh=_MESH,
    scratch_types=(
        pltpu.VMEM((3, CH), i32),
        pltpu.VMEM((CH, 16), f32),
        pltpu.VMEM((CH, 16), f32),
        pltpu.VMEM((CH, 16), f32),
        pltpu.VMEM((16,), f32),
        pltpu.VMEM_SHARED((NP, 16), f32),
        pltpu.SemaphoreType.DMA,
    ),
    compiler_params=_SC_PARAMS,
)
def _ab_kernel(pk_h, aq_h, ak_h, m_h, z16_h, ex_h, dp_h,
               pk_v, ai, aj, ex_v, m_v, acc_s, sem):
    cid = lax.axis_index("c")
    sid = lax.axis_index("s")
    wid = _worker_id()
    pltpu.sync_copy(m_h, m_v)
    pltpu.sync_copy(z16_h.at[pl.ds(sid * RPS, RPS), :],
                    acc_s.at[pl.ds(sid * RPS, RPS), :])
    plsc.subcore_barrier()
    mv = m_v[...]

    def body(i, carry):
        ch = wid + i * NW
        base = ch * CH
        pltpu.sync_copy(pk_h.at[ch], pk_v)
        d1 = pltpu.async_copy(aq_h.at[pk_v.at[0]], ai, sem)
        d2 = pltpu.async_copy(ak_h.at[pk_v.at[1]], aj, sem)
        d1.wait()
        d2.wait()

        def erow(r, c):
            a = ai[r, :] + aj[r, :]
            a = jnp.where(a >= 0, a, 0.2 * a)
            ex_v[r, :] = jnp.exp(a - mv)
            return c
        lax.fori_loop(0, CH, erow, 0, unroll=4)
        pltpu.sync_copy(ex_v, ex_h.at[pl.ds(base, CH), :])
        pltpu.sync_copy(ex_v, acc_s.at[pk_v.at[2]], add=True)
        return carry

    lax.fori_loop(0, _nchunks(wid), body, 0)
    plsc.subcore_barrier()
    pltpu.sync_copy(acc_s.at[pl.ds(sid * RPS, RPS), :],
                    dp_h.at[cid, pl.ds(sid * RPS, RPS), :])


@functools.partial(
    pl.kernel,
    out_type=jax.ShapeDtypeStruct((NC, NP, HID), f32),
    mesh=_MESH,
    scratch_types=(
        [[pltpu.VMEM((3, CH), i32),
          pltpu.VMEM((CH, 16), f32),
          pltpu.VMEM((CH, 16), f32),
          pltpu.VMEM((CH, HID), f32),
          ] for _ in range(2)],
        pltpu.VMEM_SHARED((NP, HID), f32),
        [pltpu.SemaphoreType.DMA for _ in range(3)],
    ),
    compiler_params=_SC_PARAMS,
)
def _msg_kernel(pk_h, exb_h, den_h, xr_h, z128_h, op_h, bufs, acc_s, sems):
    cid = lax.axis_index("c")
    sid = lax.axis_index("s")
    wid = _worker_id()
    pltpu.sync_copy(z128_h.at[pl.ds(sid * RPS, RPS), :],
                    acc_s.at[pl.ds(sid * RPS, RPS), :])
    plsc.subcore_barrier()

    def _stage(bu, ch):
        base = ch * CH
        pk_v, ex_v, den_v, xr_v = bu
        pltpu.sync_copy(pk_h.at[ch], pk_v)
        d1 = pltpu.async_copy(exb_h.at[pl.ds(base, CH), :], ex_v, sems[0])
        d2 = pltpu.async_copy(den_h.at[pk_v.at[2]], den_v, sems[0])
        d3 = pltpu.async_copy(xr_h.at[pk_v.at[1]], xr_v, sems[0])
        d1.wait()
        d2.wait()
        d3.wait()

    def _comp(bu):
        pk_v, ex_v, den_v, xr_v = bu

        def erow(r, c):
            at = ex_v[r, :] / (den_v[r, :] + 1e-16)
            for v in range(HID // 16):
                sl = pl.ds(v * 16, 16)
                xr_v[r, sl] = xr_v[r, sl] * at[v * 16 // 32]
            return c
        lax.fori_loop(0, CH, erow, 0, unroll=2)

    def body(i, carry):
        ba, bb = bufs[0], bufs[1]
        cha = (wid + i * NW) * 2
        chb = cha + 1
        _stage(ba, cha)
        _comp(ba)
        oda = pltpu.async_copy(ba[3], acc_s.at[ba[0].at[2]], sems[1], add=True)
        _stage(bb, chb)
        _comp(bb)
        oda.wait()
        odb = pltpu.async_copy(bb[3], acc_s.at[bb[0].at[2]], sems[2], add=True)
        odb.wait()
        return carry

    lax.fori_loop(0, _npairs(wid), body, 0)
    plsc.subcore_barrier()
    pltpu.sync_copy(acc_s.at[pl.ds(sid * RPS, RPS), :],
                    op_h.at[cid, pl.ds(sid * RPS, RPS), :])


def kernel(kg_emb, ccle, node_id, edge_index, edge_type,
           ccle_w1, ccle_b1, ccle_w2, ccle_b2,
           w_rel1, q1, k1, bias1, w_rel2, q2, k2, bias2,
           skip_w1, skip_b1, skip_w2, skip_b2):
    srcv = edge_index[0].reshape(ER, CH)
    dstv = edge_index[1].reshape(ER, CH)
    etv = edge_type.reshape(ER, CH)
    z16 = jnp.zeros((NP, 16), f32)
    z128 = jnp.zeros((NP, HID), f32)

    xin, skip = _prep(kg_emb, ccle, ccle_w1, ccle_b1, ccle_w2, ccle_b2,
                      skip_w1, skip_b1, skip_w2, skip_b2)
    pk = _pack(etv, srcv, dstv)

    xr1, aq1, ak1, pm1 = _tables(xin, w_rel1, q1, k1)
    ex1, dp1 = _ab_kernel(pk, aq1.reshape(R * N, 16),
                          ak1.reshape(R * N, 16), _mvec(pm1), z16)
    den1 = _add2(dp1)
    op1 = _msg_kernel(pk, ex1, den1, xr1.reshape(R * N, HID), z128)

    xr2, aq2, ak2, pm2 = _tables_mid(op1, bias1, w_rel2, q2, k2)
    ex2, dp2 = _ab_kernel(pk, aq2.reshape(R * N, 16),
                          ak2.reshape(R * N, 16), _mvec(pm2), z16)
    den2 = _add2(dp2)
    op2 = _msg_kernel(pk, ex2, den2, xr2.reshape(R * N, HID), z128)

    return _final(op2, bias2, skip)


# speedup vs baseline: 16.1568x; 1.0075x over previous
import functools

import jax
import jax.numpy as jnp
from jax import lax
from jax.experimental import pallas as pl
from jax.experimental.pallas import tpu as pltpu
from jax.experimental.pallas import tpu_sc as plsc

N = 10000
E = 320000
R = 8
H = 4
KG_DIM = 128
IN_CH = 256
HID = 128
OUT = 128

NC = 2
NS = 16
NW = NC * NS
CH = 128
ER = E // CH
NCHUNK1 = E // CH
NPAIR = E // (2 * CH)
NFP = NPAIR // NW
XTP = NPAIR - NFP * NW
NFULL1 = NCHUNK1 // NW
XTRA1 = NCHUNK1 - NFULL1 * NW
NP = 10240
RPS = NP // NS

_B = 1000
_NB = N // _B

f32 = jnp.float32
i32 = jnp.int32


def _leaky(x, s):
    return jnp.where(x >= 0, x, s * x)


def _prep_body(kg_ref, ccle_ref, w1_ref, b1_ref, w2_ref, b2_ref,
               sw1_ref, sb1_ref, sw2_ref, sb2_ref, xin_ref, skip_ref):
    h = jnp.dot(ccle_ref[...], w1_ref[...], preferred_element_type=f32) + b1_ref[...]
    h = _leaky(h, 0.01)
    co = jnp.dot(h, w2_ref[...], preferred_element_type=f32) + b2_ref[...]
    kg = kg_ref[...]
    xin_ref[:, :KG_DIM] = kg
    xin_ref[:, KG_DIM:] = co
    xi = jnp.concatenate([kg, co], axis=1)
    s = jnp.dot(xi, sw1_ref[...], preferred_element_type=f32) + sb1_ref[...]
    s = _leaky(s, 0.01)
    skip_ref[...] = jnp.dot(s, sw2_ref[...], preferred_element_type=f32) + sb2_ref[...]


def _prep(kg, ccle, w1, b1, w2, b2, sw1, sb1, sw2, sb2):
    full = lambda shape: pl.BlockSpec(shape, lambda j: tuple(0 for _ in shape))
    return pl.pallas_call(
        _prep_body,
        grid=(_NB,),
        in_specs=[
            pl.BlockSpec((_B, KG_DIM), lambda j: (j, 0)),
            pl.BlockSpec((_B, 4), lambda j: (j, 0)),
            full((4, 32)), full((32,)), full((32, 128)), full((128,)),
            full((IN_CH, HID)), full((HID,)), full((HID, OUT)), full((OUT,)),
        ],
        out_specs=[
            pl.BlockSpec((_B, IN_CH), lambda j: (j, 0)),
            pl.BlockSpec((_B, OUT), lambda j: (j, 0)),
        ],
        out_shape=[
            jax.ShapeDtypeStruct((N, IN_CH), f32),
            jax.ShapeDtypeStruct((N, OUT), f32),
        ],
    )(kg, ccle, w1, b1, w2, b2, sw1, sb1, sw2, sb2)


def _emit_tables(xb, q_ref, k_ref, xr_ref, aq_ref, ak_ref, pm_ref):
    xr_ref[0] = xb
    aq = jnp.dot(xb, q_ref[...], preferred_element_type=f32)
    ak = jnp.dot(xb, k_ref[...], preferred_element_type=f32)
    pad = jnp.zeros((xb.shape[0], 16 - H), f32)
    aq_ref[0] = jnp.concatenate([aq, pad], axis=1)
    ak_ref[0] = jnp.concatenate([ak, pad], axis=1)
    pm_ref[...] = jnp.concatenate(
        [jnp.max(aq, axis=0), jnp.max(ak, axis=0)]).reshape(1, 1, 2 * H)


def _tables_body(x_ref, wr_ref, q_ref, k_ref, xr_ref, aq_ref, ak_ref, pm_ref):
    xb = jnp.dot(x_ref[...], wr_ref[0], preferred_element_type=f32)
    _emit_tables(xb, q_ref, k_ref, xr_ref, aq_ref, ak_ref, pm_ref)


def _tables_mid_body(p_ref, b_ref, wr_ref, q_ref, k_ref,
                     xr_ref, aq_ref, ak_ref, pm_ref):
    x = _leaky(p_ref[0] + p_ref[1] + b_ref[...], 0.01)
    xb = jnp.dot(x, wr_ref[0], preferred_element_type=f32)
    _emit_tables(xb, q_ref, k_ref, xr_ref, aq_ref, ak_ref, pm_ref)


def _table_out_specs():
    return dict(
        out_specs=[
            pl.BlockSpec((1, _B, HID), lambda r, j: (r, j, 0)),
            pl.BlockSpec((1, _B, 16), lambda r, j: (r, j, 0)),
            pl.BlockSpec((1, _B, 16), lambda r, j: (r, j, 0)),
            pl.BlockSpec((1, 1, 2 * H), lambda r, j: (r * _NB + j, 0, 0)),
        ],
        out_shape=[
            jax.ShapeDtypeStruct((R, N, HID), f32),
            jax.ShapeDtypeStruct((R, N, 16), f32),
            jax.ShapeDtypeStruct((R, N, 16), f32),
            jax.ShapeDtypeStruct((R * _NB, 1, 2 * H), f32),
        ],
    )


def _tables(x, wr, q, k):
    in_ch = x.shape[1]
    return pl.pallas_call(
        _tables_body,
        grid=(R, _NB),
        in_specs=[
            pl.BlockSpec((_B, in_ch), lambda r, j: (j, 0)),
            pl.BlockSpec((1, in_ch, HID), lambda r, j: (r, 0, 0)),
            pl.BlockSpec((HID, H), lambda r, j: (0, 0)),
            pl.BlockSpec((HID, H), lambda r, j: (0, 0)),
        ],
        **_table_out_specs(),
    )(x, wr, q, k)


def _tables_mid(p, b, wr, q, k):
    return pl.pallas_call(
        _tables_mid_body,
        grid=(R, _NB),
        in_specs=[
            pl.BlockSpec((NC, _B, HID), lambda r, j: (0, j, 0)),
            pl.BlockSpec((HID,), lambda r, j: (0,)),
            pl.BlockSpec((1, HID, HID), lambda r, j: (r, 0, 0)),
            pl.BlockSpec((HID, H), lambda r, j: (0, 0)),
            pl.BlockSpec((HID, H), lambda r, j: (0, 0)),
        ],
        **_table_out_specs(),
    )(p, b, wr, q, k)


def _mvec(pm):
    mx = jnp.max(pm.reshape(-1, 2 * H), axis=0)
    m4 = _leaky(mx[:H] + mx[H:], 0.2)
    return jnp.concatenate([m4, jnp.zeros((16 - H,), f32)])


def _add2_body(p_ref, o_ref):
    o_ref[...] = p_ref[0] + p_ref[1]


def _add2(p):
    bp = NP // 10
    return pl.pallas_call(
        _add2_body,
        grid=(10,),
        in_specs=[pl.BlockSpec((NC, bp, 16), lambda j: (0, j, 0))],
        out_specs=pl.BlockSpec((bp, 16), lambda j: (j, 0)),
        out_shape=jax.ShapeDtypeStruct((NP, 16), f32),
    )(p)


def _pack_body(et_ref, src_ref, dst_ref, o_ref):
    e = et_ref[...] * N
    o_ref[:, 0, :] = e + dst_ref[...]
    o_ref[:, 1, :] = e + src_ref[...]
    o_ref[:, 2, :] = dst_ref[...]


def _pack(et, srcv, dstv):
    return pl.pallas_call(
        _pack_body,
        grid=(1,),
        in_specs=[pl.BlockSpec((ER, CH), lambda j: (0, 0)) for _ in range(3)],
        out_specs=pl.BlockSpec((ER, 3, CH), lambda j: (0, 0, 0)),
        out_shape=jax.ShapeDtypeStruct((ER, 3, CH), i32),
    )(et, srcv, dstv)


def _final_body(p_ref, b_ref, skip_ref, o_ref):
    o_ref[...] = _leaky(p_ref[0] + p_ref[1] + b_ref[...] + skip_ref[...], 0.01)


def _final(p, b, skip):
    return pl.pallas_call(
        _final_body,
        grid=(_NB,),
        in_specs=[
            pl.BlockSpec((NC, _B, OUT), lambda j: (0, j, 0)),
            pl.BlockSpec((OUT,), lambda j: (0,)),
            pl.BlockSpec((_B, OUT), lambda j: (j, 0)),
        ],
        out_specs=pl.BlockSpec((_B, OUT), lambda j: (j, 0)),
        out_shape=jax.ShapeDtypeStruct((N, OUT), f32),
    )(p, b, skip)


_MESH = plsc.VectorSubcoreMesh(core_axis_name="c", subcore_axis_name="s")
_SC_PARAMS = pltpu.CompilerParams(use_tc_tiling_on_sc=False)


def _worker_id():
    return lax.axis_index("s") * NC + lax.axis_index("c")


def _nchunks(wid):
    return jnp.where(wid < XTRA1, NFULL1 + 1, NFULL1).astype(i32)


def _npairs(wid):
    return jnp.where(wid < XTP, NFP + 1, NFP).astype(i32)


@functools.partial(
    pl.kernel,
    out_type=(
        jax.ShapeDtypeStruct((E, 16), f32),
        jax.ShapeDtypeStruct((NC, NP, 16), f32),
    ),
    mesh=_MESH,
    scratch_types=(
        [[pltpu.VMEM((3, CH), i32),
          pltpu.VMEM((CH, 16), f32),
          pltpu.VMEM((CH, 16), f32),
          pltpu.VMEM((CH, 16), f32),
          ] for _ in range(2)],
        pltpu.VMEM((16,), f32),
        pltpu.VMEM_SHARED((NP, 16), f32),
        [pltpu.SemaphoreType.DMA for _ in range(3)],
    ),
    compiler_params=_SC_PARAMS,
)
def _ab_kernel(pk_h, aq_h, ak_h, m_h, z16_h, ex_h, dp_h, bufs, m_v, acc_s,
               sems):
    cid = lax.axis_index("c")
    sid = lax.axis_index("s")
    wid = _worker_id()
    pltpu.sync_copy(m_h, m_v)
    pltpu.sync_copy(z16_h.at[pl.ds(sid * RPS, RPS), :],
                    acc_s.at[pl.ds(sid * RPS, RPS), :])
    plsc.subcore_barrier()
    mv = m_v[...]

    def _stage(bu, ch):
        pk_v = bu[0]
        pltpu.sync_copy(pk_h.at[ch], pk_v)
        d1 = pltpu.async_copy(aq_h.at[pk_v.at[0]], bu[1], sems[0])
        d2 = pltpu.async_copy(ak_h.at[pk_v.at[1]], bu[2], sems[0])
        d1.wait()
        d2.wait()

    def _comp(bu):
        ai, aj, ex_v = bu[1], bu[2], bu[3]

        def erow(r, c):
            x = ai[r, :] + aj[r, :]
            x = jnp.where(x >= 0, x, 0.2 * x)
            ex_v[r, :] = jnp.exp(x - mv)
            return c
        lax.fori_loop(0, CH, erow, 0, unroll=4)

    def body(i, carry):
        ba, bb = bufs[0], bufs[1]
        cha = (wid + i * NW) * 2
        chb = cha + 1
        _stage(ba, cha)
        _comp(ba)
        oa1 = pltpu.async_copy(ba[3], ex_h.at[pl.ds(cha * CH, CH), :], sems[1])
        oa2 = pltpu.async_copy(ba[3], acc_s.at[ba[0].at[2]], sems[2], add=True)
        _stage(bb, chb)
        _comp(bb)
        oa1.wait()
        oa2.wait()
        pltpu.sync_copy(bb[3], ex_h.at[pl.ds(chb * CH, CH), :])
        pltpu.sync_copy(bb[3], acc_s.at[bb[0].at[2]], add=True)
        return carry

    lax.fori_loop(0, _npairs(wid), body, 0)
    plsc.subcore_barrier()
    pltpu.sync_copy(acc_s.at[pl.ds(sid * RPS, RPS), :],
                    dp_h.at[cid, pl.ds(sid * RPS, RPS), :])


@functools.partial(
    pl.kernel,
    out_type=jax.ShapeDtypeStruct((NC, NP, HID), f32),
    mesh=_MESH,
    scratch_types=(
        [[pltpu.VMEM((3, CH), i32),
          pltpu.VMEM((CH, 16), f32),
          pltpu.VMEM((CH, 16), f32),
          pltpu.VMEM((CH, HID), f32),
          ] for _ in range(2)],
        pltpu.VMEM_SHARED((NP, HID), f32),
        [pltpu.SemaphoreType.DMA for _ in range(3)],
    ),
    compiler_params=_SC_PARAMS,
)
def _msg_kernel(pk_h, exb_h, den_h, xr_h, z128_h, op_h, bufs, acc_s, sems):
    cid = lax.axis_index("c")
    sid = lax.axis_index("s")
    wid = _worker_id()
    pltpu.sync_copy(z128_h.at[pl.ds(sid * RPS, RPS), :],
                    acc_s.at[pl.ds(sid * RPS, RPS), :])
    plsc.subcore_barrier()

    def _stage(bu, ch):
        base = ch * CH
        pk_v, ex_v, den_v, xr_v = bu
        pltpu.sync_copy(pk_h.at[ch], pk_v)
        d1 = pltpu.async_copy(exb_h.at[pl.ds(base, CH), :], ex_v, sems[0])
        d2 = pltpu.async_copy(den_h.at[pk_v.at[2]], den_v, sems[0])
        d3 = pltpu.async_copy(xr_h.at[pk_v.at[1]], xr_v, sems[0])
        d1.wait()
        d2.wait()
        d3.wait()

    def _comp(bu):
        pk_v, ex_v, den_v, xr_v = bu

        def erow(r, c):
            at = ex_v[r, :] / (den_v[r, :] + 1e-16)
            for v in range(HID // 16):
                sl = pl.ds(v * 16, 16)
                xr_v[r, sl] = xr_v[r, sl] * at[v * 16 // 32]
            return c
        lax.fori_loop(0, CH, erow, 0, unroll=2)

    def body(i, carry):
        ba, bb = bufs[0], bufs[1]
        cha = (wid + i * NW) * 2
        chb = cha + 1
        _stage(ba, cha)
        _comp(ba)
        oda = pltpu.async_copy(ba[3], acc_s.at[ba[0].at[2]], sems[1], add=True)
        _stage(bb, chb)
        _comp(bb)
        oda.wait()
        odb = pltpu.async_copy(bb[3], acc_s.at[bb[0].at[2]], sems[2], add=True)
        odb.wait()
        return carry

    lax.fori_loop(0, _npairs(wid), body, 0)
    plsc.subcore_barrier()
    pltpu.sync_copy(acc_s.at[pl.ds(sid * RPS, RPS), :],
                    op_h.at[cid, pl.ds(sid * RPS, RPS), :])


def kernel(kg_emb, ccle, node_id, edge_index, edge_type,
           ccle_w1, ccle_b1, ccle_w2, ccle_b2,
           w_rel1, q1, k1, bias1, w_rel2, q2, k2, bias2,
           skip_w1, skip_b1, skip_w2, skip_b2):
    srcv = edge_index[0].reshape(ER, CH)
    dstv = edge_index[1].reshape(ER, CH)
    etv = edge_type.reshape(ER, CH)
    z16 = jnp.zeros((NP, 16), f32)
    z128 = jnp.zeros((NP, HID), f32)

    xin, skip = _prep(kg_emb, ccle, ccle_w1, ccle_b1, ccle_w2, ccle_b2,
                      skip_w1, skip_b1, skip_w2, skip_b2)
    pk = _pack(etv, srcv, dstv)

    xr1, aq1, ak1, pm1 = _tables(xin, w_rel1, q1, k1)
    ex1, dp1 = _ab_kernel(pk, aq1.reshape(R * N, 16),
                          ak1.reshape(R * N, 16), _mvec(pm1), z16)
    den1 = _add2(dp1)
    op1 = _msg_kernel(pk, ex1, den1, xr1.reshape(R * N, HID), z128)

    xr2, aq2, ak2, pm2 = _tables_mid(op1, bias1, w_rel2, q2, k2)
    ex2, dp2 = _ab_kernel(pk, aq2.reshape(R * N, 16),
                          ak2.reshape(R * N, 16), _mvec(pm2), z16)
    den2 = _add2(dp2)
    op2 = _msg_kernel(pk, ex2, den2, xr2.reshape(R * N, HID), z128)

    return _final(op2, bias2, skip)
